# baseline (speedup 1.0000x reference)
import functools

import jax
import jax.numpy as jnp
from jax import lax
from jax.experimental import pallas as pl
from jax.experimental.pallas import tpu as pltpu
from jax.experimental.pallas import tpu_sc as plsc

_N_VOCAB = 100000
_PAD = 0
_SMOOTHING = 0.1
_CONF = 1.0 - _SMOOTHING
_S = _SMOOTHING / (_N_VOCAB - 2)
_C_ROW = (_N_VOCAB - 2) * _S * float(jnp.log(_S)) + _CONF * float(jnp.log(_CONF))

_ROWS_BLK = 8

_NC = 2
_NS = 16
_NW = _NC * _NS
_LANES = 16


def _rowsum_kernel(p_ref, t_ref, out_ref):
    i = pl.program_id(0)
    p = p_ref[...]
    t = t_ref[...]
    rowsum = jnp.sum(p, axis=1, keepdims=True)
    p0 = p[:, 0:1]
    valid = (t != _PAD).astype(jnp.float32)
    contrib = valid * (_C_ROW - _S * (rowsum - p0))
    partial = jnp.sum(contrib, axis=(0, 1), keepdims=True)

    @pl.when(i == 0)
    def _init():
        out_ref[...] = jnp.zeros_like(out_ref)

    out_ref[...] += partial


def _combine_kernel(a_ref, g_ref, out_ref):
    gsum = jnp.sum(g_ref[...], axis=(0, 1), keepdims=True)
    out_ref[...] = a_ref[...] - (_CONF - _S) * gsum


@functools.partial(
    pl.kernel,
    mesh=plsc.VectorSubcoreMesh(core_axis_name="c", subcore_axis_name="s"),
    out_type=jax.ShapeDtypeStruct((_NW, _LANES), jnp.float32),
    scratch_types=[
        pltpu.VMEM((1024 // _NW,), jnp.int32),
        pltpu.VMEM((1024 // _NW,), jnp.int32),
        pltpu.VMEM((1024 // _NW,), jnp.float32),
        pltpu.VMEM((_LANES,), jnp.float32),
        pltpu.SemaphoreType.DMA,
    ],
)
def _sc_gather(pred_hbm, t_hbm, out_hbm, t_v, idx_v, g_v, ps_v, sem):
    rpw = 1024 // _NW
    wid = lax.axis_index("s") * _NC + lax.axis_index("c")
    base = wid * rpw
    pltpu.sync_copy(t_hbm.at[pl.ds(base, rpw)], t_v)
    for k in range(rpw // _LANES):
        tv = t_v[pl.ds(k * _LANES, _LANES)]
        rows = (base + k * _LANES) + lax.broadcasted_iota(jnp.int32, (_LANES,), 0)
        idx_v[pl.ds(k * _LANES, _LANES)] = rows * jnp.int32(_N_VOCAB) + tv
    pltpu.async_copy(pred_hbm.at[idx_v], g_v, sem).wait()
    acc = jnp.zeros((_LANES,), jnp.float32)
    for k in range(rpw // _LANES):
        tv = t_v[pl.ds(k * _LANES, _LANES)]
        gv = g_v[pl.ds(k * _LANES, _LANES)]
        acc = acc + jnp.where(tv != _PAD, gv, 0.0)
    ps_v[...] = acc
    pltpu.sync_copy(ps_v, out_hbm.at[wid])


def kernel(predicts, target):
    n, v = predicts.shape
    t32 = target.astype(jnp.int32)
    t2 = t32.reshape(n, 1)

    a = pl.pallas_call(
        _rowsum_kernel,
        grid=(n // _ROWS_BLK,),
        in_specs=[
            pl.BlockSpec((_ROWS_BLK, v), lambda i: (i, 0)),
            pl.BlockSpec((_ROWS_BLK, 1), lambda i: (i, 0)),
        ],
        out_specs=pl.BlockSpec((1, 1), lambda i: (0, 0)),
        out_shape=jax.ShapeDtypeStruct((1, 1), jnp.float32),
    )(predicts, t2)

    g = _sc_gather(predicts.reshape(-1), t32)

    loss = pl.pallas_call(
        _combine_kernel,
        in_specs=[
            pl.BlockSpec((1, 1), lambda: (0, 0)),
            pl.BlockSpec((_NW, _LANES), lambda: (0, 0)),
        ],
        out_specs=pl.BlockSpec((1, 1), lambda: (0, 0)),
        out_shape=jax.ShapeDtypeStruct((1, 1), jnp.float32),
    )(a, g)
    return loss[0, 0]


# speedup vs baseline: 2.0741x; 2.0741x over previous
import math

import jax
import jax.numpy as jnp
from jax.experimental import pallas as pl
from jax.experimental.pallas import tpu as pltpu

_N_VOCAB = 100000
_PAD = 0
_SMOOTHING = 0.1
_CONF = 1.0 - _SMOOTHING
_S = _SMOOTHING / (_N_VOCAB - 2)
_C_ROW = (_N_VOCAB - 2) * _S * math.log(_S) + _CONF * math.log(_CONF)

_ROWS_BLK = 8


def _loss_kernel(t_smem, t_vmem, p_ref, out_ref):
    i = pl.program_id(0)
    p = p_ref[...]
    rowsum = jnp.sum(p, axis=1, keepdims=True)
    p0 = p[:, 0:1]

    g = jnp.zeros((1, 1), jnp.float32)
    for r in range(_ROWS_BLK):
        t_r = t_smem[r, 0]
        start = pl.multiple_of((t_r // 128) * 128, 128)
        win = p_ref[pl.ds(r, 1), pl.ds(start, 128)]
        lane = jax.lax.broadcasted_iota(jnp.int32, (1, 128), 1)
        val = jnp.sum(jnp.where(lane == (t_r % 128), win, 0.0),
                      axis=(0, 1), keepdims=True)
        g += jnp.where(t_r != _PAD, val, 0.0)

    valid = (t_vmem[...] != _PAD).astype(jnp.float32)
    contrib = valid * (_C_ROW - _S * (rowsum - p0))
    partial = jnp.sum(contrib, axis=(0, 1), keepdims=True)
    partial = partial - (_CONF - _S) * g

    @pl.when(i == 0)
    def _init():
        out_ref[...] = jnp.zeros_like(out_ref)

    out_ref[...] += partial


def kernel(predicts, target):
    n, v = predicts.shape
    t2 = target.reshape(n, 1).astype(jnp.int32)
    out = pl.pallas_call(
        _loss_kernel,
        grid=(n // _ROWS_BLK,),
        in_specs=[
            pl.BlockSpec((_ROWS_BLK, 1), lambda i: (i, 0),
                         memory_space=pltpu.SMEM),
            pl.BlockSpec((_ROWS_BLK, 1), lambda i: (i, 0)),
            pl.BlockSpec((_ROWS_BLK, v), lambda i: (i, 0)),
        ],
        out_specs=pl.BlockSpec((1, 1), lambda i: (0, 0)),
        out_shape=jax.ShapeDtypeStruct((1, 1), jnp.float32),
    )(t2, t2, predicts)
    return out[0, 0]


# speedup vs baseline: 2.3295x; 1.1231x over previous
import math

import jax
import jax.numpy as jnp
from jax.experimental import pallas as pl
from jax.experimental.pallas import tpu as pltpu

_N_VOCAB = 100000
_PAD = 0
_SMOOTHING = 0.1
_CONF = 1.0 - _SMOOTHING
_S = _SMOOTHING / (_N_VOCAB - 2)
_C_ROW = (_N_VOCAB - 2) * _S * math.log(_S) + _CONF * math.log(_CONF)

_ROWS_BLK = 16


def _loss_kernel(t_smem, t_vmem, p_ref, out_ref):
    i = pl.program_id(0)
    p = p_ref[...]
    rowsum = jnp.sum(p, axis=1, keepdims=True)
    p0 = p[:, 0:1]

    g = jnp.zeros((1, 1), jnp.float32)
    for r in range(_ROWS_BLK):
        t_r = t_smem[r, 0]
        start = pl.multiple_of((t_r // 128) * 128, 128)
        win = p_ref[pl.ds(r, 1), pl.ds(start, 128)]
        lane = jax.lax.broadcasted_iota(jnp.int32, (1, 128), 1)
        val = jnp.sum(jnp.where(lane == (t_r % 128), win, 0.0),
                      axis=(0, 1), keepdims=True)
        g += jnp.where(t_r != _PAD, val, 0.0)

    valid = (t_vmem[...] != _PAD).astype(jnp.float32)
    contrib = valid * (_C_ROW - _S * (rowsum - p0))
    partial = jnp.sum(contrib, axis=(0, 1), keepdims=True)
    partial = partial - (_CONF - _S) * g

    @pl.when(i == 0)
    def _init():
        out_ref[...] = jnp.zeros_like(out_ref)

    out_ref[...] += partial


def kernel(predicts, target):
    n, v = predicts.shape
    t2 = target.reshape(n, 1).astype(jnp.int32)
    out = pl.pallas_call(
        _loss_kernel,
        grid=(n // _ROWS_BLK,),
        in_specs=[
            pl.BlockSpec((_ROWS_BLK, 1), lambda i: (i, 0),
                         memory_space=pltpu.SMEM),
            pl.BlockSpec((_ROWS_BLK, 1), lambda i: (i, 0)),
            pl.BlockSpec((_ROWS_BLK, v), lambda i: (i, 0)),
        ],
        out_specs=pl.BlockSpec((1, 1), lambda i: (0, 0)),
        out_shape=jax.ShapeDtypeStruct((1, 1), jnp.float32),
    )(t2, t2, predicts)
    return out[0, 0]


# speedup vs baseline: 2.3863x; 1.0244x over previous
import math

import jax
import jax.numpy as jnp
from jax.experimental import pallas as pl
from jax.experimental.pallas import tpu as pltpu

_N_VOCAB = 100000
_PAD = 0
_SMOOTHING = 0.1
_CONF = 1.0 - _SMOOTHING
_S = _SMOOTHING / (_N_VOCAB - 2)
_C_ROW = (_N_VOCAB - 2) * _S * math.log(_S) + _CONF * math.log(_CONF)

_ROWS_BLK = 32


def _loss_kernel(t_smem, t_vmem, p_ref, out_ref):
    i = pl.program_id(0)
    p = p_ref[...]
    rowsum = jnp.sum(p, axis=1, keepdims=True)
    p0 = p[:, 0:1]

    g = jnp.zeros((1, 1), jnp.float32)
    for r in range(_ROWS_BLK):
        t_r = t_smem[r, 0]
        start = pl.multiple_of((t_r // 128) * 128, 128)
        win = p_ref[pl.ds(r, 1), pl.ds(start, 128)]
        lane = jax.lax.broadcasted_iota(jnp.int32, (1, 128), 1)
        val = jnp.sum(jnp.where(lane == (t_r % 128), win, 0.0),
                      axis=(0, 1), keepdims=True)
        g += jnp.where(t_r != _PAD, val, 0.0)

    valid = (t_vmem[...] != _PAD).astype(jnp.float32)
    contrib = valid * (_C_ROW - _S * (rowsum - p0))
    partial = jnp.sum(contrib, axis=(0, 1), keepdims=True)
    partial = partial - (_CONF - _S) * g

    @pl.when(i == 0)
    def _init():
        out_ref[...] = jnp.zeros_like(out_ref)

    out_ref[...] += partial


def kernel(predicts, target):
    n, v = predicts.shape
    t2 = target.reshape(n, 1).astype(jnp.int32)
    out = pl.pallas_call(
        _loss_kernel,
        grid=(n // _ROWS_BLK,),
        in_specs=[
            pl.BlockSpec((_ROWS_BLK, 1), lambda i: (i, 0),
                         memory_space=pltpu.SMEM),
            pl.BlockSpec((_ROWS_BLK, 1), lambda i: (i, 0)),
            pl.BlockSpec((_ROWS_BLK, v), lambda i: (i, 0)),
        ],
        out_specs=pl.BlockSpec((1, 1), lambda i: (0, 0)),
        out_shape=jax.ShapeDtypeStruct((1, 1), jnp.float32),
    )(t2, t2, predicts)
    return out[0, 0]
